# baseline (speedup 1.0000x reference)
import functools

import jax
import jax.numpy as jnp
from jax import lax
from jax.experimental import pallas as pl
from jax.experimental.pallas import tpu as pltpu
from jax.experimental.pallas import tpu_sc as plsc

NBUF = 8
PAIR = 2


@functools.lru_cache(maxsize=None)
def _build(batch: int, hist: int, d: int):
  info = plsc.get_sparse_core_info()
  nc, ns = info.num_cores, info.num_subcores
  nw = nc * ns
  assert PAIR * hist <= 128
  hist_pad = ((hist + 7) // 8) * 8
  nblk = batch // nw // PAIR
  assert nblk * nw * PAIR == batch
  rounds = nblk // NBUF
  assert rounds * NBUF == nblk and rounds >= 3

  mesh = plsc.VectorSubcoreMesh(core_axis_name="c", subcore_axis_name="s")

  @functools.partial(
      pl.kernel,
      out_type=jax.ShapeDtypeStruct((batch, hist_pad, 128), jnp.float32),
      mesh=mesh,
      compiler_params=pltpu.CompilerParams(use_tc_tiling_on_sc=False),
      scratch_types=[
          pltpu.VMEM((nblk, PAIR * hist), jnp.int32),
          pltpu.VMEM((NBUF, PAIR * hist, 64), jnp.float32),
          [pltpu.SemaphoreType.DMA] * NBUF,
          [pltpu.SemaphoreType.DMA] * NBUF,
      ],
  )
  def gather_kernel(idx_hbm, tab_hbm, out_hbm, idx_v, rows, gsem, wsem):
    wid = lax.axis_index("s") * nc + lax.axis_index("c")
    base = wid * nblk * PAIR
    pltpu.sync_copy(idx_hbm.at[wid], idx_v)

    def fire_gather(g, s):
      pltpu.async_copy(tab_hbm.at[idx_v.at[g]], rows.at[s], gsem[s])

    def wait_gather(s):
      pltpu.make_async_copy(
          tab_hbm.at[idx_v.at[0]], rows.at[s], gsem[s]).wait()

    def fire_write(g, s):
      for p in range(PAIR):
        pltpu.async_copy(
            rows.at[s, pl.ds(p * hist, hist)],
            out_hbm.at[base + g * PAIR + p, pl.ds(0, hist), pl.ds(0, d)],
            wsem[s])

    def wait_write(s):
      for p in range(PAIR):
        pltpu.make_async_copy(
            rows.at[s, pl.ds(p * hist, hist)],
            out_hbm.at[base, pl.ds(0, hist), pl.ds(0, d)],
            wsem[s]).wait()

    def step(g, s, first=False, ahead=True):
      s3 = (s + NBUF - 1) % NBUF
      if ahead:
        if not first:
          wait_write(s3)
        fire_gather(g + NBUF - 1, s3)
      wait_gather(s)
      fire_write(g, s)

    for s in range(NBUF - 1):
      fire_gather(s, s)

    step(0, 0, first=True)
    for s in range(1, NBUF):
      step(s, s)

    def round_body(r, carry):
      g0 = r * NBUF
      for s in range(NBUF):
        step(g0 + s, s)
      return carry

    lax.fori_loop(1, rounds - 1, round_body, 0)

    g0 = (rounds - 1) * NBUF
    step(g0, 0)
    for s in range(1, NBUF):
      step(g0 + s, s, ahead=False)
    for s in range(NBUF):
      wait_write(s)

  return gather_kernel, nw, nblk


def kernel(input, C):
  idx = input.astype(jnp.int32)
  batch, hist = idx.shape
  d = C.shape[1]
  gather_kernel, nw, nblk = _build(batch, hist, d)
  idx3 = idx.reshape(nw, nblk, PAIR * hist)
  out = gather_kernel(idx3, C)
  return out[:, :hist, :d]
